# Initial kernel scaffold; baseline (speedup 1.0000x reference)
import jax
import jax.numpy as jnp
from jax.experimental import pallas as pl


def kernel(x):
    raise NotImplementedError("write your pallas kernel here")


# speedup vs baseline: 2.4356x; 2.4356x over previous
import functools

import jax
import jax.numpy as jnp
from jax import lax
from jax.experimental import pallas as pl
from jax.experimental.pallas import tpu as pltpu
from jax.experimental.pallas import tpu_sc as plsc

R = 128
N = 32768
K = 64
L = 16
NSL = N // L
NW = 32
ROWS_PER_W = R // NW
CAP = 16384

_mesh = plsc.VectorSubcoreMesh(core_axis_name="c", subcore_axis_name="s")


def _zero_hist(hist, nbins):
    zeros = jnp.zeros((L,), jnp.int32)

    def zh(b, _):
        hist[pl.ds(b * L, L)] = zeros
        return 0

    lax.fori_loop(0, nbins, zh, 0)


def _scan_hist(hist, start_cum, topbin):
    def cond(st):
        b, cum = st
        return jnp.logical_and(cum < K, b >= 0)

    def body(st):
        b, cum = st
        hrow = hist[pl.ds(b * L, L)]
        return b - 1, cum + jnp.sum(hrow)

    b, cum = lax.while_loop(cond, body, (jnp.int32(topbin), start_cum))
    bj = b + 1
    mj = jnp.sum(hist[pl.ds(bj * L, L)])
    c_hi = cum - mj
    return bj, mj, c_hi


@functools.partial(
    pl.kernel,
    out_type=jax.ShapeDtypeStruct((R, N), jnp.float32),
    mesh=_mesh,
    compiler_params=pltpu.CompilerParams(needs_layout_passes=False),
    scratch_types=[
        pltpu.VMEM((N,), jnp.float32),
        pltpu.VMEM((CAP + 16,), jnp.int32),
        pltpu.VMEM((CAP + 16,), jnp.int32),
        pltpu.VMEM((256 * L,), jnp.int32),
    ],
)
def _topk_mask(x_hbm, out_hbm, row_v, cand_a, cand_b, hist):
    wid = lax.axis_index("s") * 2 + lax.axis_index("c")
    lane = lax.iota(jnp.int32, L)
    ones = jnp.ones((L,), jnp.int32)

    def per_row(rr, _):
        row_idx = wid * ROWS_PER_W + rr
        pltpu.sync_copy(x_hbm.at[row_idx], row_v)

        _zero_hist(hist, 256)

        def p1(i, _):
            v = row_v[pl.ds(i * L, L)]
            vr = jnp.maximum(v, 0.0)
            bits = lax.bitcast_convert_type(vr, jnp.int32)
            bn = lax.shift_right_logical(bits, 23)
            plsc.addupdate_scatter(hist, [bn * L + lane], ones)
            return 0

        lax.fori_loop(0, NSL, p1, 0)
        b0, m0, c_hi0 = _scan_hist(hist, jnp.int32(0), 255)

        def trivial(_):
            one = jnp.int32(1)
            return jnp.int32(0), one, one

        def refine(_):
            def p2(i, off):
                v = row_v[pl.ds(i * L, L)]
                vr = jnp.maximum(v, 0.0)
                bits = lax.bitcast_convert_type(vr, jnp.int32)
                bn = lax.shift_right_logical(bits, 23)
                offc = jnp.minimum(off, CAP)
                msk = jnp.logical_and(bn == b0, off < CAP)
                plsc.store_compressed(cand_a.at[pl.ds(offc, L)], bits, mask=msk)
                return off + jnp.sum(msk.astype(jnp.int32))

            m = lax.fori_loop(0, NSL, p2, jnp.int32(0))
            c_hi = c_hi0
            prefix = b0 << 23
            m_ties = m
            for li, (sh, mkv) in enumerate([(15, 255), (7, 255), (0, 127)]):
                src, dst = (cand_a, cand_b) if li % 2 == 0 else (cand_b, cand_a)
                _zero_hist(hist, mkv + 1)
                nsl = (m + L - 1) // L

                def ph(i, _, src=src, sh=sh, mkv=mkv, m=m):
                    base = i * L
                    bits = src[pl.ds(base, L)]
                    valid = (base + lane) < m
                    key = lax.shift_right_logical(bits, sh) & mkv
                    plsc.addupdate_scatter(hist, [key * L + lane], ones,
                                           mask=valid)
                    return 0

                lax.fori_loop(0, nsl, ph, 0)
                bj, mj, c_hi = _scan_hist(hist, c_hi, mkv)
                prefix = prefix | (bj << sh)
                m_ties = mj
                if li < 2:
                    def pf(i, off, src=src, dst=dst, sh=sh, mkv=mkv, m=m,
                           bj=bj):
                        base = i * L
                        bits = src[pl.ds(base, L)]
                        valid = (base + lane) < m
                        key = lax.shift_right_logical(bits, sh) & mkv
                        offc = jnp.minimum(off, CAP)
                        msk = jnp.logical_and(valid, key == bj)
                        plsc.store_compressed(dst.at[pl.ds(offc, L)], bits,
                                              mask=msk)
                        return off + jnp.sum(msk.astype(jnp.int32))

                    m = lax.fori_loop(0, nsl, pf, jnp.int32(0))
            return prefix, m_ties, K - c_hi

        t_bits, m_ties, r = lax.cond(b0 == 0, trivial, refine, 0)

        t_vec = lax.bitcast_convert_type(jnp.full((L,), t_bits, jnp.int32), jnp.float32)

        def simple(_):
            def po(i, _):
                v = row_v[pl.ds(i * L, L)]
                vr = jnp.maximum(v, 0.0)
                row_v[pl.ds(i * L, L)] = jnp.where(vr >= t_vec, vr, 0.0)
                return 0

            lax.fori_loop(0, NSL, po, 0)
            return 0

        def with_ties(_):
            def pt(i, cnt):
                v = row_v[pl.ds(i * L, L)]
                vr = jnp.maximum(v, 0.0)
                eq = vr == t_vec
                gt = vr > t_vec
                cs = plsc.cumsum(eq.astype(jnp.int32))
                keep = jnp.logical_and(eq, (cnt + cs) <= r)
                row_v[pl.ds(i * L, L)] = jnp.where(
                    jnp.logical_or(gt, keep), vr, 0.0)
                return cnt + jnp.sum(eq.astype(jnp.int32))

            lax.fori_loop(0, NSL, pt, jnp.int32(0))
            return 0

        lax.cond(m_ties == r, simple, with_ties, 0)

        pltpu.sync_copy(row_v, out_hbm.at[row_idx])
        return 0

    lax.fori_loop(0, ROWS_PER_W, per_row, 0)


def kernel(x):
    return _topk_mask(x)


# speedup vs baseline: 6.8262x; 2.8027x over previous
import functools

import jax
import jax.numpy as jnp
from jax import lax
from jax.experimental import pallas as pl
from jax.experimental.pallas import tpu as pltpu
from jax.experimental.pallas import tpu_sc as plsc

R = 128
N = 32768
K = 64
L = 16
NSL = N // L
NW = 32
ROWS_PER_W = R // NW
CAP = 8192

_mesh = plsc.VectorSubcoreMesh(core_axis_name="c", subcore_axis_name="s")


def _scan_hist(hist, start_cum, topbin):
    def cond(st):
        b, cum = st
        return jnp.logical_and(cum < K, b >= 0)

    def body(st):
        b, cum = st
        hrow = hist[pl.ds(b * L, L)]
        return b - 1, cum + jnp.sum(hrow)

    b, cum = lax.while_loop(cond, body, (topbin, start_cum))
    bj = b + 1
    mj = jnp.sum(hist[pl.ds(bj * L, L)])
    c_hi = cum - mj
    return bj, mj, c_hi


@functools.partial(
    pl.kernel,
    out_type=jax.ShapeDtypeStruct((R, N), jnp.float32),
    mesh=_mesh,
    compiler_params=pltpu.CompilerParams(needs_layout_passes=False),
    scratch_types=[
        pltpu.VMEM((N,), jnp.float32),
        pltpu.VMEM((N,), jnp.float32),
        pltpu.VMEM((CAP + 16,), jnp.int32),
        pltpu.VMEM((CAP + 16,), jnp.int32),
        pltpu.VMEM((256 * L,), jnp.int32),
        pltpu.SemaphoreType.DMA,
        pltpu.SemaphoreType.DMA,
        pltpu.SemaphoreType.DMA,
        pltpu.SemaphoreType.DMA,
    ],
)
def _topk_mask(x_hbm, out_hbm, row_a, row_b, cand_a, cand_b, hist,
               sem_ia, sem_ib, sem_oa, sem_ob):
    wid = lax.axis_index("s") * 2 + lax.axis_index("c")
    row0 = wid * ROWS_PER_W
    lane = lax.iota(jnp.int32, L)
    ones = jnp.ones((L,), jnp.int32)
    zf = jnp.zeros((L,), jnp.float32)
    zi = jnp.zeros((L,), jnp.int32)

    def select_row(row_v, mid_hook):
        def zh(b, _):
            hist[pl.ds(b * L, L)] = zi
            return 0

        lax.fori_loop(0, 256, zh, 0)

        def p1(i, mx):
            v = row_v[pl.ds(i * L, L)]
            vr = jnp.maximum(v, 0.0)
            bits = lax.bitcast_convert_type(vr, jnp.int32)
            bn = lax.shift_right_logical(bits, 23)
            plsc.addupdate_scatter(hist, [(bn << 4) + lane], ones)
            return jnp.maximum(mx, vr)

        mx = plsc.parallel_loop(0, NSL, 1, unroll=8, carry=zf)(p1)
        if mid_hook is not None:
            mid_hook()
        mxbn = lax.shift_right_logical(
            lax.bitcast_convert_type(mx, jnp.int32), 23)
        b_start = jnp.max(mxbn)
        b0, m0, c_hi0 = _scan_hist(hist, jnp.int32(0), b_start)

        def trivial(_):
            one = jnp.int32(1)
            return jnp.int32(0), one, one

        def refine(_):
            capv = jnp.full((L,), CAP, jnp.int32)

            def p2(i, off_vec):
                v = row_v[pl.ds(i * L, L)]
                vr = jnp.maximum(v, 0.0)
                bits = lax.bitcast_convert_type(vr, jnp.int32)
                bn = lax.shift_right_logical(bits, 23)
                msk = bn == b0
                idx = off_vec + plsc.cumsum(msk.astype(jnp.int32)) - 1
                msk = jnp.logical_and(msk, idx < capv)
                plsc.store_scatter(cand_a, [idx], bits, mask=msk)
                return off_vec + plsc.all_reduce_population_count(msk)

            off = plsc.parallel_loop(0, NSL, 1, unroll=8, carry=zi)(p2)
            m = jnp.max(off)
            c_hi = c_hi0
            prefix = b0 << 23
            m_ties = m
            for li, (sh, mkv) in enumerate([(15, 255), (7, 255), (0, 127)]):
                src, dst = (cand_a, cand_b) if li % 2 == 0 else (cand_b, cand_a)

                def zh2(b, _):
                    hist[pl.ds(b * L, L)] = zi
                    return 0

                lax.fori_loop(0, mkv + 1, zh2, 0)
                nsl = (m + L - 1) // L

                def ph(i, _, src=src, sh=sh, mkv=mkv, m=m):
                    base = i * L
                    bits = src[pl.ds(base, L)]
                    valid = (base + lane) < m
                    key = lax.shift_right_logical(bits, sh) & mkv
                    plsc.addupdate_scatter(hist, [(key << 4) + lane], ones,
                                           mask=valid)
                    return 0

                lax.fori_loop(0, nsl, ph, 0)
                bj, mj, c_hi = _scan_hist(hist, c_hi, jnp.int32(mkv))
                prefix = prefix | (bj << sh)
                m_ties = mj
                if li < 2:
                    def pf(i, off_vec, src=src, dst=dst, sh=sh, mkv=mkv,
                           m=m, bj=bj):
                        base = i * L
                        bits = src[pl.ds(base, L)]
                        valid = (base + lane) < m
                        key = lax.shift_right_logical(bits, sh) & mkv
                        msk = jnp.logical_and(valid, key == bj)
                        idx = off_vec + plsc.cumsum(msk.astype(jnp.int32)) - 1
                        msk = jnp.logical_and(msk, idx < capv)
                        plsc.store_scatter(dst, [idx], bits, mask=msk)
                        return off_vec + plsc.all_reduce_population_count(msk)

                    offj = lax.fori_loop(0, nsl, pf, zi)
                    m = jnp.max(offj)
            return prefix, m_ties, K - c_hi

        return lax.cond(b0 == 0, trivial, refine, 0)

    def output_row(row_v, t_bits, m_ties, r):
        t_vec = lax.bitcast_convert_type(
            jnp.full((L,), t_bits, jnp.int32), jnp.float32)

        def simple(_):
            def po(i):
                v = row_v[pl.ds(i * L, L)]
                vr = jnp.maximum(v, 0.0)
                row_v[pl.ds(i * L, L)] = jnp.where(vr >= t_vec, vr, 0.0)

            plsc.parallel_loop(0, NSL, 1, unroll=8)(po)
            return 0

        def with_ties(_):
            r_vec = jnp.full((L,), r, jnp.int32)

            def pt(i, cnt_vec):
                v = row_v[pl.ds(i * L, L)]
                vr = jnp.maximum(v, 0.0)
                eq = vr == t_vec
                gt = vr > t_vec
                cs = plsc.cumsum(eq.astype(jnp.int32))
                keep = jnp.logical_and(eq, (cnt_vec + cs) <= r_vec)
                row_v[pl.ds(i * L, L)] = jnp.where(
                    jnp.logical_or(gt, keep), vr, 0.0)
                return cnt_vec + plsc.all_reduce_population_count(eq)

            lax.fori_loop(0, NSL, pt, zi)
            return 0

        lax.cond(m_ties == r, simple, with_ties, 0)

    cin0 = pltpu.async_copy(x_hbm.at[row0 + 0], row_a, sem_ia)
    cin1 = pltpu.async_copy(x_hbm.at[row0 + 1], row_b, sem_ib)

    cin0.wait()
    t0, mt0, r0 = select_row(row_a, None)
    output_row(row_a, t0, mt0, r0)
    cout0 = pltpu.async_copy(row_a, out_hbm.at[row0 + 0], sem_oa)

    cin1.wait()
    hooked = {}

    def hook1():
        cout0.wait()
        hooked["cin2"] = pltpu.async_copy(x_hbm.at[row0 + 2], row_a, sem_ia)

    t1, mt1, r1 = select_row(row_b, hook1)
    output_row(row_b, t1, mt1, r1)
    cout1 = pltpu.async_copy(row_b, out_hbm.at[row0 + 1], sem_ob)

    hooked["cin2"].wait()

    def hook2():
        cout1.wait()
        hooked["cin3"] = pltpu.async_copy(x_hbm.at[row0 + 3], row_b, sem_ib)

    t2, mt2, r2 = select_row(row_a, hook2)
    output_row(row_a, t2, mt2, r2)
    cout2 = pltpu.async_copy(row_a, out_hbm.at[row0 + 2], sem_oa)

    hooked["cin3"].wait()
    t3, mt3, r3 = select_row(row_b, None)
    output_row(row_b, t3, mt3, r3)
    cout3 = pltpu.async_copy(row_b, out_hbm.at[row0 + 3], sem_ob)

    cout2.wait()
    cout3.wait()


def kernel(x):
    return _topk_mask(x)


# speedup vs baseline: 9.3028x; 1.3628x over previous
import functools

import jax
import jax.numpy as jnp
from jax import lax
from jax.experimental import pallas as pl
from jax.experimental.pallas import tpu as pltpu
from jax.experimental.pallas import tpu_sc as plsc

R = 128
N = 32768
K = 64
L = 16
NSL = N // L
NW = 32
ROWS_PER_W = R // NW
CAP = 8192

_mesh = plsc.VectorSubcoreMesh(core_axis_name="c", subcore_axis_name="s")


def _scan_hist(hist, start_cum, topbin):
    def cond(st):
        b, cum = st
        return jnp.logical_and(cum < K, b >= 0)

    def body(st):
        b, cum = st
        hrow = hist[pl.ds(b * L, L)]
        return b - 1, cum + jnp.sum(hrow)

    b, cum = lax.while_loop(cond, body, (topbin, start_cum))
    bj = b + 1
    mj = jnp.sum(hist[pl.ds(bj * L, L)])
    c_hi = cum - mj
    return bj, mj, c_hi


@functools.partial(
    pl.kernel,
    out_type=jax.ShapeDtypeStruct((R, N), jnp.float32),
    mesh=_mesh,
    compiler_params=pltpu.CompilerParams(needs_layout_passes=False),
    scratch_types=[
        pltpu.VMEM((N,), jnp.float32),
        pltpu.VMEM((N,), jnp.float32),
        pltpu.VMEM((CAP + 16,), jnp.int32),
        pltpu.VMEM((CAP + 16,), jnp.int32),
        pltpu.VMEM((512 * L,), jnp.int32),
        pltpu.SemaphoreType.DMA,
        pltpu.SemaphoreType.DMA,
        pltpu.SemaphoreType.DMA,
        pltpu.SemaphoreType.DMA,
    ],
)
def _topk_mask(x_hbm, out_hbm, row_a, row_b, cand_a, cand_b, hist,
               sem_ia, sem_ib, sem_oa, sem_ob):
    wid = lax.axis_index("s") * 2 + lax.axis_index("c")
    row0 = wid * ROWS_PER_W
    lane = lax.iota(jnp.int32, L)
    ones = jnp.ones((L,), jnp.int32)
    zf = jnp.zeros((L,), jnp.float32)
    zi = jnp.zeros((L,), jnp.int32)
    neg1 = jnp.full((L,), -1, jnp.int32)

    def select_row(row_v, mid_hook):
        def zh(b):
            hist[pl.ds(b * L, L)] = zi

        plsc.parallel_loop(0, 256, 1, unroll=8)(zh)

        def p1(i, mx):
            v = row_v[pl.ds(i * L, L)]
            bits = lax.bitcast_convert_type(v, jnp.int32)
            bn = lax.shift_right_logical(bits, 23)
            plsc.addupdate_scatter(hist, [(bn << 4) + lane], ones)
            return jnp.maximum(mx, v)

        mx = plsc.parallel_loop(0, NSL, 1, unroll=8, carry=zf)(p1)
        if mid_hook is not None:
            mid_hook()
        mxbn = lax.shift_right_logical(
            lax.bitcast_convert_type(jnp.maximum(mx, 0.0), jnp.int32), 23)
        b_start = jnp.max(mxbn)
        b0, m0, c_hi0 = _scan_hist(hist, jnp.int32(0), b_start)

        def trivial(_):
            one = jnp.int32(1)
            return jnp.int32(0), one, one

        def refine(_):
            capv = jnp.full((L,), CAP + 15, jnp.int32)

            def p2(i, offm1):
                v = row_v[pl.ds(i * L, L)]
                bits = lax.bitcast_convert_type(v, jnp.int32)
                bn = lax.shift_right_logical(bits, 23)
                msk = bn == b0
                idx = jnp.minimum(offm1 + plsc.cumsum(ones, mask=msk), capv)
                plsc.store_scatter(cand_a, [idx], bits, mask=msk)
                return offm1 + plsc.all_reduce_population_count(msk)

            offm1 = plsc.parallel_loop(0, NSL, 1, unroll=8, carry=neg1)(p2)
            m = jnp.max(offm1) + 1
            c_hi = c_hi0
            prefix = b0 << 23
            m_ties = m
            for li, (sh, mkv) in enumerate([(15, 255), (7, 255), (0, 127)]):
                src, dst = (cand_a, cand_b) if li % 2 == 0 else (cand_b, cand_a)

                def zh2(b):
                    hist[pl.ds(b * L, L)] = zi

                plsc.parallel_loop(0, mkv + 1, 1, unroll=8)(zh2)
                nsl = (m + L - 1) // L

                def ph(i, _, src=src, sh=sh, mkv=mkv, m=m):
                    base = i * L
                    bits = src[pl.ds(base, L)]
                    valid = (base + lane) < m
                    key = lax.shift_right_logical(bits, sh) & mkv
                    plsc.addupdate_scatter(hist, [(key << 4) + lane], ones,
                                           mask=valid)
                    return 0

                lax.fori_loop(0, nsl, ph, 0)
                bj, mj, c_hi = _scan_hist(hist, c_hi, jnp.int32(mkv))
                prefix = prefix | (bj << sh)
                m_ties = mj
                if li < 2:
                    def pf(i, offm1v, src=src, dst=dst, sh=sh, mkv=mkv,
                           m=m, bj=bj):
                        base = i * L
                        bits = src[pl.ds(base, L)]
                        valid = (base + lane) < m
                        key = lax.shift_right_logical(bits, sh) & mkv
                        msk = jnp.logical_and(valid, key == bj)
                        idx = jnp.minimum(
                            offm1v + plsc.cumsum(ones, mask=msk), capv)
                        plsc.store_scatter(dst, [idx], bits, mask=msk)
                        return offm1v + plsc.all_reduce_population_count(msk)

                    offj = lax.fori_loop(0, nsl, pf, neg1)
                    m = jnp.max(offj) + 1
            return prefix, m_ties, K - c_hi

        return lax.cond(b0 == 0, trivial, refine, 0)

    def output_row(row_v, t_bits, m_ties, r):
        t_vec = lax.bitcast_convert_type(
            jnp.full((L,), t_bits, jnp.int32), jnp.float32)

        def simple(_):
            def po(i):
                v = row_v[pl.ds(i * L, L)]
                row_v[pl.ds(i * L, L)] = jnp.where(v >= t_vec, v, 0.0)

            plsc.parallel_loop(0, NSL, 1, unroll=8)(po)
            return 0

        def with_ties(_):
            r_vec = jnp.full((L,), r, jnp.int32)

            def pt(i, cnt_vec):
                v = row_v[pl.ds(i * L, L)]
                eq = v == t_vec
                gt = v > t_vec
                cs = plsc.cumsum(eq.astype(jnp.int32))
                keep = jnp.logical_and(eq, (cnt_vec + cs) <= r_vec)
                row_v[pl.ds(i * L, L)] = jnp.where(
                    jnp.logical_or(gt, keep), v, 0.0)
                return cnt_vec + plsc.all_reduce_population_count(eq)

            lax.fori_loop(0, NSL, pt, zi)
            return 0

        lax.cond(m_ties == r, simple, with_ties, 0)

    cin0 = pltpu.async_copy(x_hbm.at[row0 + 0], row_a, sem_ia)
    cin1 = pltpu.async_copy(x_hbm.at[row0 + 1], row_b, sem_ib)

    cin0.wait()
    t0, mt0, r0 = select_row(row_a, None)
    output_row(row_a, t0, mt0, r0)
    cout0 = pltpu.async_copy(row_a, out_hbm.at[row0 + 0], sem_oa)

    cin1.wait()
    hooked = {}

    def hook1():
        cout0.wait()
        hooked["cin2"] = pltpu.async_copy(x_hbm.at[row0 + 2], row_a, sem_ia)

    t1, mt1, r1 = select_row(row_b, hook1)
    output_row(row_b, t1, mt1, r1)
    cout1 = pltpu.async_copy(row_b, out_hbm.at[row0 + 1], sem_ob)

    hooked["cin2"].wait()

    def hook2():
        cout1.wait()
        hooked["cin3"] = pltpu.async_copy(x_hbm.at[row0 + 3], row_b, sem_ib)

    t2, mt2, r2 = select_row(row_a, hook2)
    output_row(row_a, t2, mt2, r2)
    cout2 = pltpu.async_copy(row_a, out_hbm.at[row0 + 2], sem_oa)

    hooked["cin3"].wait()
    t3, mt3, r3 = select_row(row_b, None)
    output_row(row_b, t3, mt3, r3)
    cout3 = pltpu.async_copy(row_b, out_hbm.at[row0 + 3], sem_ob)

    cout2.wait()
    cout3.wait()


def kernel(x):
    return _topk_mask(x)


# speedup vs baseline: 10.5389x; 1.1329x over previous
import functools

import jax
import jax.numpy as jnp
from jax import lax
from jax.experimental import pallas as pl
from jax.experimental.pallas import tpu as pltpu
from jax.experimental.pallas import tpu_sc as plsc

R = 128
N = 32768
K = 64
L = 16
NSL = N // L
NW = 32
ROWS_PER_W = R // NW
CAP = 8192

_mesh = plsc.VectorSubcoreMesh(core_axis_name="c", subcore_axis_name="s")

_GDN = lax.GatherDimensionNumbers(
    offset_dims=(), collapsed_slice_dims=(0,), start_index_map=(0,))


def _pick(v, i_vec):
    return lax.gather(v, i_vec[:, None], _GDN, slice_sizes=(1,),
                      mode=lax.GatherScatterMode.PROMISE_IN_BOUNDS)


def _scan_groups(hist, stride, g_hi, cum0):
    def gsum(g):
        tv = hist[pl.ds(g * L, L)]
        for l in range(1, L):
            tv = tv + hist[pl.ds(l * stride + g * L, L)]
        return tv

    def cond(st):
        g, cum = st
        return jnp.logical_and(cum < K, g >= 0)

    def body(st):
        g, cum = st
        return g - 1, cum + jnp.sum(gsum(g))

    g, cum = lax.while_loop(cond, body, (g_hi, cum0))
    g0 = g + 1
    tv = gsum(g0)
    cum_before = cum - jnp.sum(tv)
    incl = cum_before + plsc.cumsum(lax.rev(tv, (0,)))
    i0 = plsc.all_reduce_ffs(incl >= K)
    j = 15 - i0
    mj = _pick(tv, jnp.maximum(j, 0))
    c_hi = _pick(incl, jnp.minimum(i0, 15)) - mj
    bj = j + g0 * L
    return bj, mj, c_hi


@functools.partial(
    pl.kernel,
    out_type=jax.ShapeDtypeStruct((R, N), jnp.float32),
    mesh=_mesh,
    compiler_params=pltpu.CompilerParams(needs_layout_passes=False),
    scratch_types=[
        pltpu.VMEM((N,), jnp.float32),
        pltpu.VMEM((N,), jnp.float32),
        pltpu.VMEM((CAP + 16,), jnp.int32),
        pltpu.VMEM((CAP + 16,), jnp.int32),
        pltpu.VMEM((512 * L,), jnp.int32),
        pltpu.SemaphoreType.DMA,
        pltpu.SemaphoreType.DMA,
        pltpu.SemaphoreType.DMA,
        pltpu.SemaphoreType.DMA,
    ],
)
def _topk_mask(x_hbm, out_hbm, row_a, row_b, cand_a, cand_b, hist,
               sem_ia, sem_ib, sem_oa, sem_ob):
    wid = lax.axis_index("s") * 2 + lax.axis_index("c")
    row0 = wid * ROWS_PER_W
    lane = lax.iota(jnp.int32, L)
    ones = jnp.ones((L,), jnp.int32)
    zf = jnp.zeros((L,), jnp.float32)
    zi = jnp.zeros((L,), jnp.int32)
    neg1 = jnp.full((L,), -1, jnp.int32)
    lane9 = lane << 9
    lane8 = lane << 8
    lane7 = lane << 7

    def select_row(row_v, mid_hook):
        def zh(b):
            hist[pl.ds(b * L, L)] = zi

        plsc.parallel_loop(0, 512, 1, unroll=8)(zh)

        def p1(i, mx):
            v = row_v[pl.ds(i * L, L)]
            bits = lax.bitcast_convert_type(v, jnp.int32)
            bn = lax.shift_right_logical(bits, 23)
            plsc.addupdate_scatter(hist, [lane9 + bn], ones)
            return jnp.maximum(mx, v)

        mx = plsc.parallel_loop(0, NSL, 1, unroll=8, carry=zf)(p1)
        if mid_hook is not None:
            mid_hook()
        mxbn = lax.shift_right_logical(
            lax.bitcast_convert_type(jnp.maximum(mx, 0.0), jnp.int32), 23)
        g_start = jnp.max(mxbn) // L
        b0, m0, c_hi0 = _scan_groups(hist, 512, g_start, jnp.int32(0))
        b0_sc = jnp.max(b0)

        def trivial(_):
            return zi, jnp.int32(0), zi

        def refine(_):
            capv = jnp.full((L,), CAP + 15, jnp.int32)

            def p2(i, offm1):
                v = row_v[pl.ds(i * L, L)]
                bits = lax.bitcast_convert_type(v, jnp.int32)
                bn = lax.shift_right_logical(bits, 23)
                msk = bn == b0
                idx = jnp.minimum(offm1 + plsc.cumsum(ones, mask=msk), capv)
                plsc.store_scatter(cand_a, [idx], bits, mask=msk)
                return offm1 + plsc.all_reduce_population_count(msk)

            offm1 = plsc.parallel_loop(0, NSL, 1, unroll=8, carry=neg1)(p2)
            m = jnp.max(offm1) + 1
            c_hi = c_hi0
            prefix = b0 << 23
            m_ties = m0
            for li, (sh, mkv, lmaj) in enumerate(
                    [(15, 255, lane8), (7, 255, lane8), (0, 127, lane7)]):
                src, dst = (cand_a, cand_b) if li % 2 == 0 else (cand_b, cand_a)

                def zh2(b):
                    hist[pl.ds(b * L, L)] = zi

                plsc.parallel_loop(0, L * (mkv + 1) // L, 1, unroll=8)(zh2)
                nsl = (m + L - 1) // L

                def ph(i, _, src=src, sh=sh, mkv=mkv, m=m, lmaj=lmaj):
                    base = i * L
                    bits = src[pl.ds(base, L)]
                    valid = (base + lane) < m
                    key = lax.shift_right_logical(bits, sh) & mkv
                    plsc.addupdate_scatter(hist, [lmaj + key], ones,
                                           mask=valid)
                    return 0

                lax.fori_loop(0, nsl, ph, 0)
                c_hi_sc = jnp.max(c_hi)
                bj, mj, c_hi = _scan_groups(
                    hist, mkv + 1, jnp.int32((mkv + 1) // L - 1), c_hi_sc)
                prefix = prefix | (bj << sh)
                m_ties = mj
                if li < 2:
                    def pf(i, offm1v, src=src, dst=dst, sh=sh, mkv=mkv,
                           m=m, bj=bj):
                        base = i * L
                        bits = src[pl.ds(base, L)]
                        valid = (base + lane) < m
                        key = lax.shift_right_logical(bits, sh) & mkv
                        msk = jnp.logical_and(valid, key == bj)
                        idx = jnp.minimum(
                            offm1v + plsc.cumsum(ones, mask=msk), capv)
                        plsc.store_scatter(dst, [idx], bits, mask=msk)
                        return offm1v + plsc.all_reduce_population_count(msk)

                    offj = lax.fori_loop(0, nsl, pf, neg1)
                    m = jnp.max(offj) + 1
            r_vec = K - c_hi
            tie_diff = jnp.max(m_ties - r_vec)
            return prefix, tie_diff, r_vec

        return lax.cond(b0_sc <= 0, trivial, refine, 0)

    def output_row(row_v, t_bits_vec, tie_diff, r_vec):
        t_vec = lax.bitcast_convert_type(t_bits_vec, jnp.float32)

        def simple(_):
            def po(i):
                v = row_v[pl.ds(i * L, L)]
                row_v[pl.ds(i * L, L)] = jnp.where(v >= t_vec, v, 0.0)

            plsc.parallel_loop(0, NSL, 1, unroll=8)(po)
            return 0

        def with_ties(_):
            def pt(i, cnt_vec):
                v = row_v[pl.ds(i * L, L)]
                eq = v == t_vec
                gt = v > t_vec
                cs = plsc.cumsum(eq.astype(jnp.int32))
                keep = jnp.logical_and(eq, (cnt_vec + cs) <= r_vec)
                row_v[pl.ds(i * L, L)] = jnp.where(
                    jnp.logical_or(gt, keep), v, 0.0)
                return cnt_vec + plsc.all_reduce_population_count(eq)

            lax.fori_loop(0, NSL, pt, zi)
            return 0

        lax.cond(tie_diff == 0, simple, with_ties, 0)

    cin0 = pltpu.async_copy(x_hbm.at[row0 + 0], row_a, sem_ia)
    cin1 = pltpu.async_copy(x_hbm.at[row0 + 1], row_b, sem_ib)

    cin0.wait()
    t0, d0, r0 = select_row(row_a, None)
    output_row(row_a, t0, d0, r0)
    cout0 = pltpu.async_copy(row_a, out_hbm.at[row0 + 0], sem_oa)

    cin1.wait()
    hooked = {}

    def hook1():
        cout0.wait()
        hooked["cin2"] = pltpu.async_copy(x_hbm.at[row0 + 2], row_a, sem_ia)

    t1, d1, r1 = select_row(row_b, hook1)
    output_row(row_b, t1, d1, r1)
    cout1 = pltpu.async_copy(row_b, out_hbm.at[row0 + 1], sem_ob)

    hooked["cin2"].wait()

    def hook2():
        cout1.wait()
        hooked["cin3"] = pltpu.async_copy(x_hbm.at[row0 + 3], row_b, sem_ib)

    t2, d2, r2 = select_row(row_a, hook2)
    output_row(row_a, t2, d2, r2)
    cout2 = pltpu.async_copy(row_a, out_hbm.at[row0 + 2], sem_oa)

    hooked["cin3"].wait()
    t3, d3, r3 = select_row(row_b, None)
    output_row(row_b, t3, d3, r3)
    cout3 = pltpu.async_copy(row_b, out_hbm.at[row0 + 3], sem_ob)

    cout2.wait()
    cout3.wait()


def kernel(x):
    return _topk_mask(x)


# speedup vs baseline: 10.6254x; 1.0082x over previous
import functools

import jax
import jax.numpy as jnp
from jax import lax
from jax.experimental import pallas as pl
from jax.experimental.pallas import tpu as pltpu
from jax.experimental.pallas import tpu_sc as plsc

R = 128
N = 32768
K = 64
L = 16
NSL = N // L
NW = 32
ROWS_PER_W = R // NW
CAP = 8192

_mesh = plsc.VectorSubcoreMesh(core_axis_name="c", subcore_axis_name="s")

_GDN = lax.GatherDimensionNumbers(
    offset_dims=(), collapsed_slice_dims=(0,), start_index_map=(0,))


def _pick(v, i_vec):
    return lax.gather(v, i_vec[:, None], _GDN, slice_sizes=(1,),
                      mode=lax.GatherScatterMode.PROMISE_IN_BOUNDS)


def _scan_groups(hist, stride, g_hi, cum0):
    def gsum(g):
        tv = hist[pl.ds(g * L, L)]
        for l in range(1, L):
            tv = tv + hist[pl.ds(l * stride + g * L, L)]
        return tv

    def cond(st):
        g, cum = st
        return jnp.logical_and(cum < K, g >= 0)

    def body(st):
        g, cum = st
        return g - 1, cum + jnp.sum(gsum(g))

    g, cum = lax.while_loop(cond, body, (g_hi, cum0))
    g0 = g + 1
    tv = gsum(g0)
    cum_before = cum - jnp.sum(tv)
    incl = cum_before + plsc.cumsum(lax.rev(tv, (0,)))
    i0 = plsc.all_reduce_ffs(incl >= K)
    j = 15 - i0
    mj = _pick(tv, jnp.maximum(j, 0))
    c_hi = _pick(incl, jnp.minimum(i0, 15)) - mj
    bj = j + g0 * L
    return bj, mj, c_hi


@functools.partial(
    pl.kernel,
    out_type=jax.ShapeDtypeStruct((R, N), jnp.float32),
    mesh=_mesh,
    compiler_params=pltpu.CompilerParams(needs_layout_passes=False),
    scratch_types=[
        pltpu.VMEM((N,), jnp.float32),
        pltpu.VMEM((N,), jnp.float32),
        pltpu.VMEM((CAP + 16,), jnp.int32),
        pltpu.VMEM((CAP + 16,), jnp.int32),
        pltpu.VMEM((512 * L,), jnp.int32),
        pltpu.SemaphoreType.DMA,
        pltpu.SemaphoreType.DMA,
        pltpu.SemaphoreType.DMA,
        pltpu.SemaphoreType.DMA,
    ],
)
def _topk_mask(x_hbm, out_hbm, row_a, row_b, cand_a, cand_b, hist,
               sem_ia, sem_ib, sem_oa, sem_ob):
    wid = lax.axis_index("s") * 2 + lax.axis_index("c")
    row0 = wid * ROWS_PER_W
    lane = lax.iota(jnp.int32, L)
    ones = jnp.ones((L,), jnp.int32)
    zf = jnp.zeros((L,), jnp.float32)
    zi = jnp.zeros((L,), jnp.int32)
    neg1 = jnp.full((L,), -1, jnp.int32)
    lane9 = lane << 9
    lane8 = lane << 8
    lane7 = lane << 7

    def zero_hist():
        def zh(b):
            off = ((b >> 4) << 9) + ((b & 15) << 4)
            hist[pl.ds(off, L)] = zi

        plsc.parallel_loop(0, 256, 1, unroll=8)(zh)

    def p1_pass(row_v):
        def p1(i, mx):
            v = row_v[pl.ds(i * L, L)]
            bits = lax.bitcast_convert_type(v, jnp.int32)
            bn = lax.shift_right_logical(bits, 23)
            plsc.addupdate_scatter(hist, [lane9 + bn], ones)
            return jnp.maximum(mx, v)

        return plsc.parallel_loop(0, NSL, 1, unroll=8, carry=zf)(p1)

    def select_rest(row_v, mx, dma_hook):
        mxbn = lax.shift_right_logical(
            lax.bitcast_convert_type(jnp.maximum(mx, 0.0), jnp.int32), 23)
        g_start = jnp.max(mxbn) // L
        b0, m0, c_hi0 = _scan_groups(hist, 512, g_start, jnp.int32(0))
        b0_sc = jnp.max(b0)
        capv = jnp.full((L,), CAP + 15, jnp.int32)

        def p2(i, offm1):
            v = row_v[pl.ds(i * L, L)]
            bits = lax.bitcast_convert_type(v, jnp.int32)
            bn = lax.shift_right_logical(bits, 23)
            msk = bn == b0
            idx = jnp.minimum(offm1 + plsc.cumsum(ones, mask=msk), capv)
            plsc.store_scatter(cand_a, [idx], bits, mask=msk)
            return offm1 + plsc.all_reduce_population_count(msk)

        offm1 = plsc.parallel_loop(0, NSL, 1, unroll=8, carry=neg1)(p2)
        if dma_hook is not None:
            dma_hook()
        m_in = jnp.max(offm1) + 1

        def trivial(_):
            return zi, jnp.int32(0), zi

        def refine(_):
            m = m_in
            c_hi = c_hi0
            prefix = b0 << 23
            m_ties = m0
            for li, (sh, mkv, lmaj) in enumerate(
                    [(15, 255, lane8), (7, 255, lane8), (0, 127, lane7)]):
                src, dst = (cand_a, cand_b) if li % 2 == 0 else (cand_b, cand_a)

                def zh2(b):
                    hist[pl.ds(b * L, L)] = zi

                plsc.parallel_loop(0, L * (mkv + 1) // L, 1, unroll=8)(zh2)
                nsl = (m + L - 1) // L

                def ph(i, _, src=src, sh=sh, mkv=mkv, m=m, lmaj=lmaj):
                    base = i * L
                    bits = src[pl.ds(base, L)]
                    valid = (base + lane) < m
                    key = lax.shift_right_logical(bits, sh) & mkv
                    plsc.addupdate_scatter(hist, [lmaj + key], ones,
                                           mask=valid)
                    return 0

                lax.fori_loop(0, nsl, ph, 0)
                c_hi_sc = jnp.max(c_hi)
                bj, mj, c_hi = _scan_groups(
                    hist, mkv + 1, jnp.int32((mkv + 1) // L - 1), c_hi_sc)
                prefix = prefix | (bj << sh)
                m_ties = mj
                if li < 2:
                    def pf(i, offm1v, src=src, dst=dst, sh=sh, mkv=mkv,
                           m=m, bj=bj):
                        base = i * L
                        bits = src[pl.ds(base, L)]
                        valid = (base + lane) < m
                        key = lax.shift_right_logical(bits, sh) & mkv
                        msk = jnp.logical_and(valid, key == bj)
                        idx = jnp.minimum(
                            offm1v + plsc.cumsum(ones, mask=msk), capv)
                        plsc.store_scatter(dst, [idx], bits, mask=msk)
                        return offm1v + plsc.all_reduce_population_count(msk)

                    offj = lax.fori_loop(0, nsl, pf, neg1)
                    m = jnp.max(offj) + 1
            r_vec = K - c_hi
            tie_diff = jnp.max(m_ties - r_vec)
            return prefix, tie_diff, r_vec

        return lax.cond(b0_sc <= 0, trivial, refine, 0)

    def fused_out_p1(rowX, sel, rowY):
        t_bits_vec, tie_diff, r_vec = sel
        t_vec = lax.bitcast_convert_type(t_bits_vec, jnp.float32)

        def simple(_):
            def fo(i, mx):
                v = rowX[pl.ds(i * L, L)]
                rowX[pl.ds(i * L, L)] = jnp.where(v >= t_vec, v, 0.0)
                w = rowY[pl.ds(i * L, L)]
                bits = lax.bitcast_convert_type(w, jnp.int32)
                bn = lax.shift_right_logical(bits, 23)
                plsc.addupdate_scatter(hist, [lane9 + bn], ones)
                return jnp.maximum(mx, w)

            return plsc.parallel_loop(0, NSL, 1, unroll=8, carry=zf)(fo)

        def with_ties(_):
            def ft(i, car):
                cnt_vec, mx = car
                v = rowX[pl.ds(i * L, L)]
                eq = v == t_vec
                gt = v > t_vec
                cs = plsc.cumsum(eq.astype(jnp.int32))
                keep = jnp.logical_and(eq, (cnt_vec + cs) <= r_vec)
                rowX[pl.ds(i * L, L)] = jnp.where(
                    jnp.logical_or(gt, keep), v, 0.0)
                w = rowY[pl.ds(i * L, L)]
                bits = lax.bitcast_convert_type(w, jnp.int32)
                bn = lax.shift_right_logical(bits, 23)
                plsc.addupdate_scatter(hist, [lane9 + bn], ones)
                return (cnt_vec + plsc.all_reduce_population_count(eq),
                        jnp.maximum(mx, w))

            _, mx = plsc.parallel_loop(0, NSL, 1, unroll=8,
                                       carry=(zi, zf))(ft)
            return mx

        return lax.cond(tie_diff == 0, simple, with_ties, 0)

    def output_row(row_v, sel):
        t_bits_vec, tie_diff, r_vec = sel
        t_vec = lax.bitcast_convert_type(t_bits_vec, jnp.float32)

        def simple(_):
            def po(i):
                v = row_v[pl.ds(i * L, L)]
                row_v[pl.ds(i * L, L)] = jnp.where(v >= t_vec, v, 0.0)

            plsc.parallel_loop(0, NSL, 1, unroll=8)(po)
            return 0

        def with_ties(_):
            def pt(i, cnt_vec):
                v = row_v[pl.ds(i * L, L)]
                eq = v == t_vec
                gt = v > t_vec
                cs = plsc.cumsum(eq.astype(jnp.int32))
                keep = jnp.logical_and(eq, (cnt_vec + cs) <= r_vec)
                row_v[pl.ds(i * L, L)] = jnp.where(
                    jnp.logical_or(gt, keep), v, 0.0)
                return cnt_vec + plsc.all_reduce_population_count(eq)

            lax.fori_loop(0, NSL, pt, zi)
            return 0

        lax.cond(tie_diff == 0, simple, with_ties, 0)

    cin0 = pltpu.async_copy(x_hbm.at[row0 + 0], row_a, sem_ia)
    cin1 = pltpu.async_copy(x_hbm.at[row0 + 1], row_b, sem_ib)

    cin0.wait()
    zero_hist()
    mx0 = p1_pass(row_a)
    cin1.wait()
    sel0 = select_rest(row_a, mx0, None)
    zero_hist()
    mx1 = fused_out_p1(row_a, sel0, row_b)
    cout0 = pltpu.async_copy(row_a, out_hbm.at[row0 + 0], sem_oa)

    hooked = {}

    def hook1():
        cout0.wait()
        hooked["cin2"] = pltpu.async_copy(x_hbm.at[row0 + 2], row_a, sem_ia)

    sel1 = select_rest(row_b, mx1, hook1)
    zero_hist()
    hooked["cin2"].wait()
    mx2 = fused_out_p1(row_b, sel1, row_a)
    cout1 = pltpu.async_copy(row_b, out_hbm.at[row0 + 1], sem_ob)

    def hook2():
        cout1.wait()
        hooked["cin3"] = pltpu.async_copy(x_hbm.at[row0 + 3], row_b, sem_ib)

    sel2 = select_rest(row_a, mx2, hook2)
    zero_hist()
    hooked["cin3"].wait()
    mx3 = fused_out_p1(row_a, sel2, row_b)
    cout2 = pltpu.async_copy(row_a, out_hbm.at[row0 + 2], sem_oa)

    sel3 = select_rest(row_b, mx3, None)
    output_row(row_b, sel3)
    cout3 = pltpu.async_copy(row_b, out_hbm.at[row0 + 3], sem_ob)

    cout2.wait()
    cout3.wait()


def kernel(x):
    return _topk_mask(x)
